# baseline (speedup 1.0000x reference)
import functools
import math

import jax
import jax.numpy as jnp
from jax import lax
from jax.experimental import pallas as pl
from jax.experimental.pallas import tpu as pltpu
from jax.experimental.pallas import tpu_sc as plsc

D = 128
SCALE = math.sqrt(float(D))

NC = 2
NS = 16
NW = NC * NS
SPC = 2
CPAD = 112
NBUF = 4


def _scale_body(t_ref, o_ref):
    o_ref[...] = t_ref[...] * SCALE


def _scale_table(table):
    rows = table.shape[0]
    blk = 2000
    return pl.pallas_call(
        _scale_body,
        grid=(rows // blk,),
        in_specs=[pl.BlockSpec((blk, D), lambda i: (i, 0))],
        out_specs=pl.BlockSpec((blk, D), lambda i: (i, 0)),
        out_shape=jax.ShapeDtypeStruct((rows, D), jnp.float32),
    )(table)


def _gather_body(nchunks, seq, table_hbm, ids_hbm, out_hbm, idx_v, *scr):
    bufs = scr[:NBUF]
    gsems = scr[NBUF:2 * NBUF]
    wsems = scr[2 * NBUF:3 * NBUF]
    wid = lax.axis_index("s") * NC + lax.axis_index("c")
    pltpu.sync_copy(ids_hbm.at[wid], idx_v)
    pair_base = wid * nchunks
    nrows = (SPC * seq + 7) // 8 * 8

    def start_gather(c, b):
        pltpu.async_copy(table_hbm.at[idx_v.at[c]], bufs[b], gsems[b])

    for b in range(NBUF):
        start_gather(b, b)

    def step(g, issue_next):
        c0 = g * NBUF
        for b in range(NBUF):
            pltpu.make_async_copy(
                table_hbm.at[idx_v.at[0]], bufs[b], gsems[b]).wait()
            pltpu.async_copy(
                bufs[b].at[pl.ds(0, nrows)],
                out_hbm.at[pl.ds((pair_base + c0 + b) * CPAD, nrows)],
                wsems[b])
        for b in range(NBUF):
            pltpu.make_async_copy(
                bufs[b].at[pl.ds(0, nrows)],
                out_hbm.at[pl.ds(0, nrows)], wsems[b]).wait()
            if issue_next:
                start_gather(c0 + NBUF + b, b)

    def body(g, carry):
        step(g, True)
        return carry

    lax.fori_loop(0, nchunks // NBUF - 1, body, 0)
    step(nchunks // NBUF - 1, False)


def kernel(input_ids, table):
    nsent, seq = input_ids.shape
    npair = nsent // SPC
    nchunks = npair // NW
    ids = input_ids.reshape(NW, nchunks, SPC * seq)
    ids = jnp.pad(ids, ((0, 0), (0, 0), (0, CPAD - SPC * seq)))

    scaled = _scale_table(table)

    mesh = plsc.VectorSubcoreMesh(core_axis_name="c", subcore_axis_name="s")
    gather = pl.kernel(
        functools.partial(_gather_body, nchunks, seq),
        mesh=mesh,
        out_type=jax.ShapeDtypeStruct((npair * CPAD, D), jnp.float32),
        scratch_types=(
            [pltpu.VMEM((nchunks, CPAD), jnp.int32)]
            + [pltpu.VMEM((CPAD, D), jnp.float32) for _ in range(NBUF)]
            + [pltpu.SemaphoreType.DMA for _ in range(2 * NBUF)]
        ),
    )
    flat = gather(scaled, ids)
    return flat.reshape(npair, CPAD, D)[:, :SPC * seq, :].reshape(
        nsent, seq, D)
